# baseline (speedup 1.0000x reference)
import jax
import jax.numpy as jnp
import numpy as np
from jax.experimental import pallas as pl
from jax.experimental.pallas import tpu as pltpu

B, T, D, M = 128, 64, 256, 1024
TB = 8
NT = TB * B
NBLK = T // TB
DECAY, EPS = 0.99, 1e-05
MAXENT = np.log(M)


def _scode_nll(S, eye):
    diag_s = jnp.sum(S * eye, axis=1, keepdims=True)
    m = jnp.max(S, axis=1, keepdims=True)
    lse = m + jnp.log(jnp.sum(jnp.exp(S - m), axis=1, keepdims=True))
    return jnp.sum(diag_s - lse)


def _fused_kernel(a_ref, v_ref, emb_ref, cnt_ref, wgt_ref,
                  aq_ref, vq_ref, cm_ref, eq_ref, emb_out_ref,
                  acc_ref, wsa_ref, wsv_ref,
                  csa_ref, csv_ref, ca_ref, cv_ref,
                  emb2_ref, e2_ref):
    i = pl.program_id(0)
    emb = emb_ref[...]

    @pl.when(i == 0)
    def _precomp():
        emb2_ref[...] = emb + emb
        e2_ref[...] = jnp.sum(emb * emb, axis=1, keepdims=True).T

    emb2 = emb2_ref[...]
    e2 = e2_ref[...]

    def encode(x, y):
        x2 = jnp.sum(x * x, axis=1, keepdims=True)
        xe2 = jax.lax.dot_general(x, emb2, (((1,), (1,)), ((), ())))
        d = (e2 + x2) - xe2

        iota_m = jax.lax.broadcasted_iota(jnp.int32, (NT, M), 1)
        dmin = jnp.min(d, axis=1, keepdims=True)
        onehot = (jnp.min(jnp.where(d == dmin, iota_m, M), axis=1, keepdims=True)
                  == iota_m).astype(jnp.float32)

        ez1 = jnp.exp(jnp.sqrt(jnp.maximum(dmin, 0.0)) - jnp.sqrt(jnp.maximum(d, 0.0)))
        s = jnp.sum(ez1, axis=1, keepdims=True)
        logs = jnp.log(s)
        ez1sq = ez1 * ez1
        s2 = jnp.sum(ez1sq, axis=1, keepdims=True)

        ent = logs - jnp.sum(ez1 * jnp.log(ez1 + 1e-5 * s), axis=1, keepdims=True) / s
        adj = 1.0 - ent / MAXENT

        llog10 = jnp.log(ez1 + 1e-10 * s)

        q = jax.lax.dot_general(onehot, emb, (((1,), (0,)), ((), ())))
        w = jax.lax.dot_general(onehot, adj * (x + y),
                                (((0,), (0,)), ((), ())))
        c = jax.lax.dot_general(adj, onehot, (((0,), (0,)), ((), ())))
        return q, ez1sq, s2, logs, llog10, onehot, w, c

    x = jnp.swapaxes(a_ref[...], 0, 1).reshape(NT, D)
    y = jnp.swapaxes(v_ref[...], 0, 1).reshape(NT, D)
    qa, num05a, den05a, logsa, llog10a, oha, wa, csa = encode(x, y)
    qv, num05v, den05v, logsv, llog10v, ohv, wv, csv = encode(y, x)
    aq_ref[...] = jnp.swapaxes(qa.reshape(TB, B, D), 0, 1)
    vq_ref[...] = jnp.swapaxes(qv.reshape(TB, B, D), 0, 1)

    eye = (jax.lax.broadcasted_iota(jnp.int32, (B, B), 0)
           == jax.lax.broadcasted_iota(jnp.int32, (B, B), 1)).astype(jnp.float32)
    nll = jnp.zeros((), jnp.float32)
    for tt in range(TB):
        sl = slice(tt * B, (tt + 1) * B)
        su1 = jax.lax.dot_general(
            num05a[sl], llog10v[sl], (((1,), (1,)), ((), ())))
        nll += _scode_nll(su1 / den05a[sl] - logsv[sl].T, eye)
        su2 = jax.lax.dot_general(
            num05v[sl], llog10a[sl], (((1,), (1,)), ((), ())))
        nll += _scode_nll(su2 / den05v[sl] - logsa[sl].T, eye)
    nll = jnp.reshape(nll, (1, 1))

    cnt_a = jnp.sum(oha.reshape(TB, B, M), axis=0)
    cnt_v = jnp.sum(ohv.reshape(TB, B, M), axis=0)

    @pl.when(i == 0)
    def _init():
        acc_ref[...] = nll
        wsa_ref[...] = wa
        wsv_ref[...] = wv
        csa_ref[...] = csa
        csv_ref[...] = csv
        ca_ref[...] = cnt_a
        cv_ref[...] = cnt_v

    @pl.when(i > 0)
    def _acc():
        acc_ref[...] += nll
        wsa_ref[...] += wa
        wsv_ref[...] += wv
        csa_ref[...] += csa
        csv_ref[...] += csv
        ca_ref[...] += cnt_a
        cv_ref[...] += cnt_v

    @pl.when(i == NBLK - 1)
    def _finale():
        cm_ref[...] = acc_ref[...] * (-0.5 / (T * B))

        counts_a = ca_ref[...]
        counts_v = cv_ref[...]
        iota_b = jax.lax.broadcasted_iota(jnp.int32, (B, M), 1)
        ama = jnp.min(jnp.where(counts_a == jnp.max(counts_a, axis=1, keepdims=True),
                                iota_b, M), axis=1, keepdims=True)
        amv = jnp.min(jnp.where(counts_v == jnp.max(counts_v, axis=1, keepdims=True),
                                iota_b, M), axis=1, keepdims=True)
        eq_ref[...] = jnp.reshape(jnp.sum((ama == amv).astype(jnp.int32)), (1, 1))

        ec = DECAY * cnt_ref[...] + (1.0 - DECAY) * csv_ref[...]
        n = jnp.sum(ec)
        ec = (ec + EPS) / (n + M * EPS) * n
        ew = DECAY * wgt_ref[...] + 0.5 * (1.0 - DECAY) * wsv_ref[...]
        ec2 = DECAY * ec + (1.0 - DECAY) * csa_ref[...]
        n2 = jnp.sum(ec2)
        ec2 = (ec2 + EPS) / (n2 + M * EPS) * n2
        ew2 = DECAY * ew + 0.5 * (1.0 - DECAY) * wsa_ref[...]
        emb_out_ref[...] = ew2 / ec2.T


def kernel(audio_semantic, video_semantic, embedding, ema_count, ema_weight):
    a_q, v_q, cm, eq, new_embedding = pl.pallas_call(
        _fused_kernel,
        grid=(NBLK,),
        in_specs=[
            pl.BlockSpec((B, TB, D), lambda t: (0, t, 0)),
            pl.BlockSpec((B, TB, D), lambda t: (0, t, 0)),
            pl.BlockSpec((M, D), lambda t: (0, 0)),
            pl.BlockSpec((1, M), lambda t: (0, 0)),
            pl.BlockSpec((M, D), lambda t: (0, 0)),
        ],
        out_specs=[
            pl.BlockSpec((B, TB, D), lambda t: (0, t, 0)),
            pl.BlockSpec((B, TB, D), lambda t: (0, t, 0)),
            pl.BlockSpec((1, 1), lambda t: (0, 0)),
            pl.BlockSpec((1, 1), lambda t: (0, 0)),
            pl.BlockSpec((M, D), lambda t: (0, 0)),
        ],
        out_shape=[
            jax.ShapeDtypeStruct((B, T, D), jnp.float32),
            jax.ShapeDtypeStruct((B, T, D), jnp.float32),
            jax.ShapeDtypeStruct((1, 1), jnp.float32),
            jax.ShapeDtypeStruct((1, 1), jnp.int32),
            jax.ShapeDtypeStruct((M, D), jnp.float32),
        ],
        scratch_shapes=[
            pltpu.VMEM((1, 1), jnp.float32),
            pltpu.VMEM((M, D), jnp.float32),
            pltpu.VMEM((M, D), jnp.float32),
            pltpu.VMEM((1, M), jnp.float32),
            pltpu.VMEM((1, M), jnp.float32),
            pltpu.VMEM((B, M), jnp.float32),
            pltpu.VMEM((B, M), jnp.float32),
            pltpu.VMEM((M, D), jnp.float32),
            pltpu.VMEM((1, M), jnp.float32),
        ],
    )(audio_semantic, video_semantic, embedding,
      ema_count.reshape(1, M), ema_weight)

    return (a_q, v_q, cm[0, 0], eq[0, 0], new_embedding)
